# Initial kernel scaffold; baseline (speedup 1.0000x reference)
import jax
import jax.numpy as jnp
from jax.experimental import pallas as pl


def kernel(input_ids, emb, W1, b1, W2, b2):
    raise NotImplementedError("write your pallas kernel here")


# speedup vs baseline: 15.1698x; 15.1698x over previous
import functools

import jax
import jax.numpy as jnp
from jax import lax
from jax.experimental import pallas as pl
from jax.experimental.pallas import tpu as pltpu
from jax.experimental.pallas import tpu_sc as plsc

DPAD = 16


def _transform_body(emb_ref, w1_ref, b1_ref, w2_ref, b2_ref, out_ref):
    e = emb_ref[...]
    h = jnp.dot(e, w1_ref[...], preferred_element_type=jnp.float32)
    h = jnp.maximum(h + b1_ref[...], 0.0)
    out_ref[...] = (
        jnp.dot(h, w2_ref[...], preferred_element_type=jnp.float32) + b2_ref[...]
    )


def _transform_table(emb, W1, b1, W2p, b2p, tile):
    V, E = emb.shape
    H = W1.shape[1]
    return pl.pallas_call(
        _transform_body,
        grid=(V // tile,),
        in_specs=[
            pl.BlockSpec((tile, E), lambda i: (i, 0)),
            pl.BlockSpec((E, H), lambda i: (0, 0)),
            pl.BlockSpec((1, H), lambda i: (0, 0)),
            pl.BlockSpec((H, DPAD), lambda i: (0, 0)),
            pl.BlockSpec((1, DPAD), lambda i: (0, 0)),
        ],
        out_specs=pl.BlockSpec((tile, DPAD), lambda i: (i, 0)),
        out_shape=jax.ShapeDtypeStruct((V, DPAD), jnp.float32),
    )(emb, W1, b1.reshape(1, H), W2p, b2p.reshape(1, DPAD))


@functools.lru_cache(maxsize=None)
def _make_gather(V, N):
    info = plsc.get_sparse_core_info()
    NC, NS = info.num_cores, info.num_subcores
    NW = NC * NS
    assert N % NW == 0
    n_per_w = N // NW
    IPS = 128
    CHUNK = 5120
    assert n_per_w % CHUNK == 0 and CHUNK % (8 * IPS) == 0
    n_chunks = n_per_w // CHUNK
    n_streams = CHUNK // IPS
    mesh = plsc.VectorSubcoreMesh(core_axis_name="c", subcore_axis_name="s")

    @functools.partial(
        pl.kernel,
        mesh=mesh,
        out_type=jax.ShapeDtypeStruct((N, DPAD), jnp.float32),
        scratch_types=[
            pltpu.VMEM((n_streams, IPS), jnp.int32),
            pltpu.VMEM((CHUNK, DPAD), jnp.float32),
            pltpu.SemaphoreType.DMA,
        ],
        compiler_params=pltpu.CompilerParams(use_tc_tiling_on_sc=False),
    )
    def gather(tbl_hbm, idx_hbm, out_hbm, idx_v, rows_v, sem):
        wid = lax.axis_index("s") * NC + lax.axis_index("c")
        row0 = wid * n_per_w

        def chunk_body(i, _):
            off = pl.multiple_of(row0 + i * CHUNK, CHUNK)
            idx_row = pl.multiple_of(off // IPS, 8)
            pltpu.sync_copy(idx_hbm.at[pl.ds(idx_row, n_streams)], idx_v)

            def fire(j, _):
                pltpu.async_copy(
                    tbl_hbm.at[idx_v.at[j]],
                    rows_v.at[pl.ds(j * IPS, IPS)],
                    sem,
                )
                return 0

            lax.fori_loop(0, n_streams, fire, 0)
            pltpu.make_async_copy(
                out_hbm.at[pl.ds(off, CHUNK)], rows_v, sem
            ).wait()
            pltpu.sync_copy(rows_v, out_hbm.at[pl.ds(off, CHUNK)])
            return 0

        lax.fori_loop(0, n_chunks, chunk_body, 0)

    return gather


def kernel(input_ids, emb, W1, b1, W2, b2):
    V, E = emb.shape
    H = W1.shape[1]
    O = W2.shape[1]
    Bb, Ll = input_ids.shape
    N = Bb * Ll

    W2p = jnp.zeros((H, DPAD), jnp.float32).at[:, :O].set(W2)
    b2p = jnp.zeros((DPAD,), jnp.float32).at[:O].set(b2)

    table = _transform_table(emb, W1, b1, W2p, b2p, tile=10000)

    idx2d = input_ids.reshape(N // 128, 128).astype(jnp.int32)
    out16 = _make_gather(V, N)(table, idx2d)

    return out16[:, :O].reshape(Bb, Ll, O)


# speedup vs baseline: 17.1022x; 1.1274x over previous
import functools

import jax
import jax.numpy as jnp
from jax import lax
from jax.experimental import pallas as pl
from jax.experimental.pallas import tpu as pltpu
from jax.experimental.pallas import tpu_sc as plsc

DPAD = 16


def _transform_body(ep_ref, w1_ref, b1_ref, w2_ref, b2_ref, out_ref):
    ep = ep_ref[...]
    w1 = w1_ref[...]
    b1 = b1_ref[...]
    w2 = w2_ref[...]
    b2 = b2_ref[...]
    outs = []
    for j in range(8):
        e_j = ep[:, 32 * j : 32 * (j + 1)]
        h_j = jnp.dot(e_j, w1, preferred_element_type=jnp.float32)
        h_j = jnp.maximum(h_j + b1, 0.0)
        x_j = jnp.dot(h_j, w2, preferred_element_type=jnp.float32) + b2
        outs.append(x_j)
    out_ref[...] = jnp.concatenate(outs, axis=1)


def _transform_table_packed(ep, W1, b1, W2p, b2p, tile8):
    V8 = ep.shape[0]
    H = W1.shape[1]
    return pl.pallas_call(
        _transform_body,
        grid=(V8 // tile8,),
        in_specs=[
            pl.BlockSpec((tile8, 256), lambda i: (i, 0)),
            pl.BlockSpec((32, H), lambda i: (0, 0)),
            pl.BlockSpec((1, H), lambda i: (0, 0)),
            pl.BlockSpec((H, DPAD), lambda i: (0, 0)),
            pl.BlockSpec((1, DPAD), lambda i: (0, 0)),
        ],
        out_specs=pl.BlockSpec((tile8, 128), lambda i: (i, 0)),
        out_shape=jax.ShapeDtypeStruct((V8, 128), jnp.float32),
    )(ep, W1, b1.reshape(1, H), W2p, b2p.reshape(1, DPAD))


@functools.lru_cache(maxsize=None)
def _make_gather(V, N):
    info = plsc.get_sparse_core_info()
    NC, NS = info.num_cores, info.num_subcores
    NW = NC * NS
    assert N % NW == 0
    n_per_w = N // NW
    IPS = 128
    CHUNK = 5120
    assert n_per_w % CHUNK == 0 and CHUNK % (8 * IPS) == 0
    n_chunks = n_per_w // CHUNK
    n_streams = CHUNK // IPS
    mesh = plsc.VectorSubcoreMesh(core_axis_name="c", subcore_axis_name="s")

    @functools.partial(
        pl.kernel,
        mesh=mesh,
        out_type=jax.ShapeDtypeStruct((N, DPAD), jnp.float32),
        scratch_types=[
            pltpu.VMEM((n_streams, IPS), jnp.int32),
            pltpu.VMEM((CHUNK, DPAD), jnp.float32),
            pltpu.SemaphoreType.DMA,
        ],
        compiler_params=pltpu.CompilerParams(use_tc_tiling_on_sc=False),
    )
    def gather(tbl_hbm, idx_hbm, out_hbm, idx_v, rows_v, sem):
        wid = lax.axis_index("s") * NC + lax.axis_index("c")
        row0 = wid * n_per_w

        def chunk_body(i, _):
            off = pl.multiple_of(row0 + i * CHUNK, CHUNK)
            idx_row = pl.multiple_of(off // IPS, 8)
            pltpu.sync_copy(idx_hbm.at[pl.ds(idx_row, n_streams)], idx_v)

            def fire(j, _):
                pltpu.async_copy(
                    tbl_hbm.at[idx_v.at[j]],
                    rows_v.at[pl.ds(j * IPS, IPS)],
                    sem,
                )
                return 0

            lax.fori_loop(0, n_streams, fire, 0)
            pltpu.make_async_copy(
                out_hbm.at[pl.ds(off, CHUNK)], rows_v, sem
            ).wait()
            pltpu.sync_copy(rows_v, out_hbm.at[pl.ds(off, CHUNK)])
            return 0

        lax.fori_loop(0, n_chunks, chunk_body, 0)

    return gather


def kernel(input_ids, emb, W1, b1, W2, b2):
    V, E = emb.shape
    H = W1.shape[1]
    O = W2.shape[1]
    Bb, Ll = input_ids.shape
    N = Bb * Ll

    W2p = jnp.zeros((H, DPAD), jnp.float32).at[:, :O].set(W2)
    b2p = jnp.zeros((DPAD,), jnp.float32).at[:O].set(b2)

    ep = emb.reshape(V // 8, 256)
    packed = _transform_table_packed(ep, W1, b1, W2p, b2p, tile8=5000)
    table = packed.reshape(V, DPAD)

    idx2d = input_ids.reshape(N // 128, 128).astype(jnp.int32)
    out16 = _make_gather(V, N)(table, idx2d)

    return out16[:, :O].reshape(Bb, Ll, O)


# speedup vs baseline: 18.3786x; 1.0746x over previous
import functools

import jax
import jax.numpy as jnp
from jax import lax
from jax.experimental import pallas as pl
from jax.experimental.pallas import tpu as pltpu
from jax.experimental.pallas import tpu_sc as plsc

DPAD = 16


def _transform_body(et_ref, w1t_ref, b1c_ref, w2t_ref, b2c_ref, out_ref):
    et = et_ref[...]
    h = jnp.dot(w1t_ref[...], et, preferred_element_type=jnp.float32)
    h = jnp.maximum(h + b1c_ref[...], 0.0)
    out_ref[...] = (
        jnp.dot(w2t_ref[...], h, preferred_element_type=jnp.float32) + b2c_ref[...]
    )


def _transform_table_t(embT, W1T, b1c, W2pT, b2pc, tile):
    E, V = embT.shape
    H = W1T.shape[0]
    return pl.pallas_call(
        _transform_body,
        grid=(pl.cdiv(V, tile),),
        in_specs=[
            pl.BlockSpec((E, tile), lambda i: (0, i)),
            pl.BlockSpec((H, E), lambda i: (0, 0)),
            pl.BlockSpec((H, 1), lambda i: (0, 0)),
            pl.BlockSpec((DPAD, H), lambda i: (0, 0)),
            pl.BlockSpec((DPAD, 1), lambda i: (0, 0)),
        ],
        out_specs=pl.BlockSpec((DPAD, tile), lambda i: (0, i)),
        out_shape=jax.ShapeDtypeStruct((DPAD, V), jnp.float32),
    )(embT, W1T, b1c, W2pT, b2pc)


@functools.lru_cache(maxsize=None)
def _make_gather(V, N):
    info = plsc.get_sparse_core_info()
    NC, NS = info.num_cores, info.num_subcores
    NW = NC * NS
    assert N % NW == 0
    n_per_w = N // NW
    IPS = 128
    CHUNK = 5120
    assert n_per_w % CHUNK == 0 and CHUNK % (8 * IPS) == 0
    n_chunks = n_per_w // CHUNK
    n_streams = CHUNK // IPS
    mesh = plsc.VectorSubcoreMesh(core_axis_name="c", subcore_axis_name="s")

    @functools.partial(
        pl.kernel,
        mesh=mesh,
        out_type=jax.ShapeDtypeStruct((N, DPAD), jnp.float32),
        scratch_types=[
            pltpu.VMEM((n_streams, IPS), jnp.int32),
            pltpu.VMEM((CHUNK, DPAD), jnp.float32),
            pltpu.SemaphoreType.DMA,
        ],
        compiler_params=pltpu.CompilerParams(use_tc_tiling_on_sc=False),
    )
    def gather(tbl_hbm, idx_hbm, out_hbm, idx_v, rows_v, sem):
        wid = lax.axis_index("s") * NC + lax.axis_index("c")
        row0 = wid * n_per_w

        def chunk_body(i, _):
            off = pl.multiple_of(row0 + i * CHUNK, CHUNK)
            idx_row = pl.multiple_of(off // IPS, 8)
            pltpu.sync_copy(idx_hbm.at[pl.ds(idx_row, n_streams)], idx_v)

            def fire(j, _):
                pltpu.async_copy(
                    tbl_hbm.at[idx_v.at[j]],
                    rows_v.at[pl.ds(j * IPS, IPS)],
                    sem,
                )
                return 0

            lax.fori_loop(0, n_streams, fire, 0)
            pltpu.make_async_copy(
                out_hbm.at[pl.ds(off, CHUNK)], rows_v, sem
            ).wait()
            pltpu.sync_copy(rows_v, out_hbm.at[pl.ds(off, CHUNK)])
            return 0

        lax.fori_loop(0, n_chunks, chunk_body, 0)

    return gather


def kernel(input_ids, emb, W1, b1, W2, b2):
    V, E = emb.shape
    H = W1.shape[1]
    O = W2.shape[1]
    Bb, Ll = input_ids.shape
    N = Bb * Ll

    W2pT = jnp.zeros((DPAD, H), jnp.float32).at[:O, :].set(W2.T)
    b2pc = jnp.zeros((DPAD, 1), jnp.float32).at[:O, 0].set(b2)

    tableT = _transform_table_t(
        emb.T, W1.T, b1.reshape(H, 1), W2pT, b2pc, tile=65536
    )
    table = tableT.T

    idx2d = input_ids.reshape(N // 128, 128).astype(jnp.int32)
    out16 = _make_gather(V, N)(table, idx2d)

    return out16[:, :O].reshape(Bb, Ll, O)
